# baseline (speedup 1.0000x reference)
import jax
import jax.numpy as jnp
from jax import lax
from jax.experimental import pallas as pl
from jax.experimental.pallas import tpu as pltpu
from jax.experimental.pallas import tpu_sc as plsc

B = 16384
D = 128

_info = plsc.get_sparse_core_info()
NC = _info.num_cores
NS = _info.num_subcores
L = _info.num_lanes
NW = NC * NS
BPW = B // NW
CH = 128
NCHUNK = BPW // CH


def _body(ut, itab, us, its, out, uidx, iidx, ub, ib, outv, sem0, sem1):
    wid = lax.axis_index("s") * NC + lax.axis_index("c")
    base = wid * BPW
    pltpu.sync_copy(us.at[pl.ds(base, BPW)], uidx)
    pltpu.sync_copy(its.at[pl.ds(base, BPW)], iidx)
    sems = (sem0, sem1)

    def start(c):
        slot = c % 2
        du = pltpu.async_copy(ut.at[uidx.at[pl.ds(c * CH, CH)]], ub.at[slot],
                              sems[slot])
        di = pltpu.async_copy(itab.at[iidx.at[pl.ds(c * CH, CH)]], ib.at[slot],
                              sems[slot])
        return du, di

    lane = lax.iota(jnp.int32, L)
    pend = start(0)
    for c in range(NCHUNK):
        slot = c % 2
        du, di = pend
        du.wait()
        di.wait()
        if c + 1 < NCHUNK:
            pend = start(c + 1)
        ubs = ub.at[slot]
        ibs = ib.at[slot]

        def gbody(g, _, ubs=ubs, ibs=ibs, c=c):
            rows = lane + g * L
            accs = [jnp.zeros((L,), jnp.float32) for _ in range(4)]
            for d in range(D):
                cols = jnp.full((L,), d, jnp.int32)
                uv = plsc.load_gather(ubs, [rows, cols])
                iv = plsc.load_gather(ibs, [rows, cols])
                accs[d % 4] = accs[d % 4] + uv * iv
            acc = (accs[0] + accs[1]) + (accs[2] + accs[3])
            sig = 1.0 / (1.0 + jnp.exp(-acc))
            outv[pl.ds(c * CH + g * L, L)] = sig
            return 0

        lax.fori_loop(0, CH // L, gbody, 0)
    pltpu.sync_copy(outv, out.at[pl.ds(base, BPW)])


def kernel(users, items, user_table, item_table):
    k = pl.kernel(
        _body,
        out_type=jax.ShapeDtypeStruct((B,), jnp.float32),
        mesh=plsc.VectorSubcoreMesh(core_axis_name="c", subcore_axis_name="s"),
        compiler_params=pltpu.CompilerParams(needs_layout_passes=False),
        scratch_types=[
            pltpu.VMEM((BPW,), jnp.int32),
            pltpu.VMEM((BPW,), jnp.int32),
            pltpu.VMEM((2, CH, D), jnp.float32),
            pltpu.VMEM((2, CH, D), jnp.float32),
            pltpu.VMEM((BPW,), jnp.float32),
            pltpu.SemaphoreType.DMA,
            pltpu.SemaphoreType.DMA,
        ],
    )
    return k(user_table, item_table, users, items)


# speedup vs baseline: 2.9709x; 2.9709x over previous
import jax
import jax.numpy as jnp
from jax import lax
from jax.experimental import pallas as pl
from jax.experimental.pallas import tpu as pltpu
from jax.experimental.pallas import tpu_sc as plsc

B = 16384
D = 128

_info = plsc.get_sparse_core_info()
NC = _info.num_cores
NS = _info.num_subcores
L = _info.num_lanes
NW = NC * NS
BPW = B // NW
CH = 128
NCHUNK = BPW // CH


def _body(ut, itab, us, its, out, uidx, iidx, ub, ib, outv, stage, sem0, sem1):
    wid = lax.axis_index("s") * NC + lax.axis_index("c")
    base = wid * BPW
    pltpu.sync_copy(us.at[pl.ds(base, BPW)], uidx)
    pltpu.sync_copy(its.at[pl.ds(base, BPW)], iidx)
    sems = (sem0, sem1)

    def start(c):
        slot = c % 2
        du = pltpu.async_copy(ut.at[uidx.at[pl.ds(c * CH, CH)]], ub.at[slot],
                              sems[slot])
        di = pltpu.async_copy(itab.at[iidx.at[pl.ds(c * CH, CH)]], ib.at[slot],
                              sems[slot])
        return du, di

    lane17 = lax.iota(jnp.int32, L) * 17
    pend = start(0)
    for c in range(NCHUNK):
        slot = c % 2
        du, di = pend
        du.wait()
        di.wait()
        if c + 1 < NCHUNK:
            pend = start(c + 1)
        ubs = ub.at[slot]
        ibs = ib.at[slot]

        def gbody(g, _, ubs=ubs, ibs=ibs, c=c):
            for r in range(L):
                row = g * L + r
                ps = []
                for k in range(D // L):
                    uvk = ubs[row, pl.ds(k * L, L)]
                    ivk = ibs[row, pl.ds(k * L, L)]
                    ps.append(uvk * ivk)
                while len(ps) > 1:
                    ps = [a + b for a, b in zip(ps[::2], ps[1::2])]
                plsc.store_scatter(stage, [lane17 + r], ps[0])
            tot = []
            for j in range(L):
                tot.append(stage[pl.ds(j * 17, L)])
            while len(tot) > 1:
                tot = [a + b for a, b in zip(tot[::2], tot[1::2])]
            acc = tot[0]
            sig = 1.0 / (1.0 + jnp.exp(-acc))
            outv[pl.ds(c * CH + g * L, L)] = sig
            return 0

        lax.fori_loop(0, CH // L, gbody, 0)
    pltpu.sync_copy(outv, out.at[pl.ds(base, BPW)])


def kernel(users, items, user_table, item_table):
    k = pl.kernel(
        _body,
        out_type=jax.ShapeDtypeStruct((B,), jnp.float32),
        mesh=plsc.VectorSubcoreMesh(core_axis_name="c", subcore_axis_name="s"),
        compiler_params=pltpu.CompilerParams(needs_layout_passes=False),
        scratch_types=[
            pltpu.VMEM((BPW,), jnp.int32),
            pltpu.VMEM((BPW,), jnp.int32),
            pltpu.VMEM((2, CH, D), jnp.float32),
            pltpu.VMEM((2, CH, D), jnp.float32),
            pltpu.VMEM((BPW,), jnp.float32),
            pltpu.VMEM((L * 17,), jnp.float32),
            pltpu.SemaphoreType.DMA,
            pltpu.SemaphoreType.DMA,
        ],
    )
    return k(user_table, item_table, users, items)
